# baseline (speedup 1.0000x reference)
import jax
import jax.numpy as jnp
from jax.experimental import pallas as pl
from jax.experimental.pallas import tpu as pltpu

K = 64
G = 12
BM = 512
BN = 1536
BMD = 2048
BND = 768
BMT = 64


def _enc_kernel(x_ref, w_ref, benc_ref, bdec_ref, o_ref, m1_ref):
    xm = x_ref[...] - bdec_ref[...]
    pre = jnp.dot(xm, w_ref[...].T, preferred_element_type=jnp.float32)
    acts = jax.nn.relu(pre + benc_ref[...])
    o_ref[...] = acts
    m1_ref[...] = jnp.max(acts.reshape(acts.shape[0], G, BN // G), axis=1)


def _encode_acts(x, W_enc, b_enc, b_dec):
    M, D = x.shape
    F = W_enc.shape[0]
    return pl.pallas_call(
        _enc_kernel,
        grid=(F // BN, M // BM),
        in_specs=[
            pl.BlockSpec((BM, D), lambda j, i: (i, 0)),
            pl.BlockSpec((BN, D), lambda j, i: (j, 0)),
            pl.BlockSpec((1, BN), lambda j, i: (0, j)),
            pl.BlockSpec((1, D), lambda j, i: (0, 0)),
        ],
        out_specs=[
            pl.BlockSpec((BM, BN), lambda j, i: (i, j)),
            pl.BlockSpec((BM, BN // G), lambda j, i: (i, j)),
        ],
        out_shape=[
            jax.ShapeDtypeStruct((M, F), jnp.float32),
            jax.ShapeDtypeStruct((M, F // G), jnp.float32),
        ],
    )(x, W_enc, b_enc.reshape(1, F), b_dec.reshape(1, D))


def _thresh_kernel(a_ref, m1_ref, t_ref, m2_ref, lo_ref, hi_ref, tb_ref,
                   done_ref):
    rows = a_ref.shape[0]
    m1w = m1_ref.shape[1]

    m2_ref[...] = jnp.max(m1_ref[...].reshape(rows, 8, m1w // 8), axis=1)
    rmaxf = jnp.max(m2_ref[...], axis=1, keepdims=True)
    rmax = jax.lax.bitcast_convert_type(rmaxf, jnp.int32)

    def msearch(target):
        def body(it, t):
            cand = t | (1 << (30 - it))
            candf = jax.lax.bitcast_convert_type(cand, jnp.float32)
            cnt = jnp.sum((m2_ref[...] >= candf).astype(jnp.int32), axis=1,
                          keepdims=True)
            return jnp.where(cnt >= target, cand, t)

        return jax.lax.fori_loop(0, 31, body, jnp.zeros((rows, 1), jnp.int32))

    m65 = msearch(K + 1)

    lo = m65 + 1
    lo_ref[...] = lo
    hi_ref[...] = jnp.maximum(rmax, lo)
    tb_ref[...] = lo
    done_ref[...] = (lo > rmax).astype(jnp.int32)

    def cond(go):
        return go

    def body(_):
        lo, hi = lo_ref[...], hi_ref[...]
        done = done_ref[...] != 0
        mid = jax.lax.shift_right_logical(lo + hi + 1, 1)
        midf = jax.lax.bitcast_convert_type(mid, jnp.float32)
        cnt = jnp.sum((a_ref[...] >= midf).astype(jnp.int32), axis=1,
                      keepdims=True)
        nd = jnp.logical_not(done)
        hit = jnp.logical_and(cnt == K, nd)
        ge = cnt >= K
        nlo = jnp.where(jnp.logical_and(nd, ge), mid, lo)
        nhi = jnp.where(jnp.logical_and(nd, ge), hi, mid - 1)
        conv = nhi <= nlo
        t = tb_ref[...]
        t = jnp.where(hit, mid, t)
        t = jnp.where(jnp.logical_and(conv, jnp.logical_and(
            nd, jnp.logical_not(hit))), nlo, t)
        ndone = jnp.logical_or(done, jnp.logical_or(hit, conv))
        lo_ref[...] = nlo
        hi_ref[...] = nhi
        tb_ref[...] = t
        done_ref[...] = ndone.astype(jnp.int32)
        return jnp.logical_not(jnp.all(ndone))

    jax.lax.while_loop(cond, body, jnp.logical_not(jnp.all(done_ref[...] != 0)))
    t_ref[...] = jax.lax.bitcast_convert_type(tb_ref[...], jnp.float32)


def _thresholds(acts, m1):
    M, F = acts.shape
    bmt = min(BMT, M)
    return pl.pallas_call(
        _thresh_kernel,
        grid=(M // bmt,),
        in_specs=[
            pl.BlockSpec((bmt, F), lambda i: (i, 0)),
            pl.BlockSpec((bmt, F // G), lambda i: (i, 0)),
        ],
        out_specs=pl.BlockSpec((bmt, 1), lambda i: (i, 0)),
        out_shape=jax.ShapeDtypeStruct((M, 1), jnp.float32),
        scratch_shapes=[pltpu.VMEM((bmt, F // G // 8), jnp.float32)]
        + [pltpu.VMEM((bmt, 1), jnp.int32) for _ in range(4)],
    )(acts, m1)


def _dec_kernel(a_ref, w_ref, t_ref, bdec_ref, o_ref):
    j = pl.program_id(1)
    f = jnp.where(a_ref[...] >= t_ref[...], a_ref[...], 0.0)
    part = jax.lax.dot_general(f, w_ref[...], (((1,), (1,)), ((), ())),
                               preferred_element_type=jnp.float32)

    @pl.when(j == 0)
    def _():
        o_ref[...] = part + bdec_ref[...]

    @pl.when(j > 0)
    def _():
        o_ref[...] += part


def _decode(acts, thresh, W_dec, b_dec):
    M, F = acts.shape
    D = W_dec.shape[0]
    bmd = min(BMD, M)
    nj, ni = F // BND, M // bmd
    return pl.pallas_call(
        _dec_kernel,
        grid=(ni, nj),
        in_specs=[
            pl.BlockSpec((bmd, BND), lambda i, j: (i, j)),
            pl.BlockSpec((D, BND), lambda i, j: (0, j)),
            pl.BlockSpec((bmd, 1), lambda i, j: (i, 0)),
            pl.BlockSpec((1, D), lambda i, j: (0, 0)),
        ],
        out_specs=pl.BlockSpec((bmd, D), lambda i, j: (i, 0)),
        out_shape=jax.ShapeDtypeStruct((M, D), jnp.float32),
    )(acts, W_dec, thresh, b_dec.reshape(1, D))


def _ae(x, W_enc, b_enc, W_dec, b_dec):
    orig_shape = x.shape
    xf = x.reshape(-1, orig_shape[-1])
    acts, m1 = _encode_acts(xf, W_enc, b_enc, b_dec)
    thresh = _thresholds(acts, m1)
    recon = _decode(acts, thresh, W_dec, b_dec)
    return recon.reshape(orig_shape)


def kernel(x_attn_0, x_mlp_0,
           W_enc_attn_0, b_enc_attn_0, W_dec_attn_0, b_dec_attn_0,
           W_enc_mlp_0, b_enc_mlp_0, W_dec_mlp_0, b_dec_mlp_0):
    recon_attn = _ae(x_attn_0, W_enc_attn_0, b_enc_attn_0, W_dec_attn_0, b_dec_attn_0)
    recon_mlp = _ae(x_mlp_0, W_enc_mlp_0, b_enc_mlp_0, W_dec_mlp_0, b_dec_mlp_0)
    return (recon_attn, recon_mlp)


# speedup vs baseline: 1.1949x; 1.1949x over previous
import jax
import jax.numpy as jnp
from jax.experimental import pallas as pl
from jax.experimental.pallas import tpu as pltpu

K = 64
G = 12
BM = 512
BN = 1536
BMD = 2048
BND = 768
BMT = 256


def _enc_kernel(x_ref, w_ref, benc_ref, bdec_ref, o_ref, m1_ref):
    xm = x_ref[...] - bdec_ref[...]
    pre = jnp.dot(xm, w_ref[...].T, preferred_element_type=jnp.float32)
    acts = jax.nn.relu(pre + benc_ref[...])
    o_ref[...] = acts
    m1_ref[...] = jnp.max(acts.reshape(acts.shape[0], G, BN // G), axis=1)


def _encode_acts(x, W_enc, b_enc, b_dec):
    M, D = x.shape
    F = W_enc.shape[0]
    return pl.pallas_call(
        _enc_kernel,
        grid=(F // BN, M // BM),
        in_specs=[
            pl.BlockSpec((BM, D), lambda j, i: (i, 0)),
            pl.BlockSpec((BN, D), lambda j, i: (j, 0)),
            pl.BlockSpec((1, BN), lambda j, i: (0, j)),
            pl.BlockSpec((1, D), lambda j, i: (0, 0)),
        ],
        out_specs=[
            pl.BlockSpec((BM, BN), lambda j, i: (i, j)),
            pl.BlockSpec((BM, BN // G), lambda j, i: (i, j)),
        ],
        out_shape=[
            jax.ShapeDtypeStruct((M, F), jnp.float32),
            jax.ShapeDtypeStruct((M, F // G), jnp.float32),
        ],
    )(x, W_enc, b_enc.reshape(1, F), b_dec.reshape(1, D))


def _thresh_kernel(a_ref, m1_ref, t_ref, m2_ref, lo_ref, hi_ref, tb_ref,
                   done_ref):
    rows = a_ref.shape[0]
    m1w = m1_ref.shape[1]

    m2_ref[...] = jnp.max(m1_ref[...].reshape(rows, 8, m1w // 8), axis=1)
    rmaxf = jnp.max(m2_ref[...], axis=1, keepdims=True)
    rmax = jax.lax.bitcast_convert_type(rmaxf, jnp.int32)

    def msearch(target):
        def body(it, t):
            cand = t | (1 << (30 - it))
            candf = jax.lax.bitcast_convert_type(cand, jnp.float32)
            cnt = jnp.sum((m2_ref[...] >= candf).astype(jnp.int32), axis=1,
                          keepdims=True)
            return jnp.where(cnt >= target, cand, t)

        return jax.lax.fori_loop(0, 31, body, jnp.zeros((rows, 1), jnp.int32))

    m65 = msearch(K + 1)

    lo = m65 + 1
    lo_ref[...] = lo
    hi_ref[...] = jnp.maximum(rmax, lo)
    tb_ref[...] = lo
    done_ref[...] = (lo > rmax).astype(jnp.int32)

    def cond(go):
        return go

    def body(_):
        lo, hi = lo_ref[...], hi_ref[...]
        done = done_ref[...] != 0
        mid = jax.lax.shift_right_logical(lo + hi + 1, 1)
        midf = jax.lax.bitcast_convert_type(mid, jnp.float32)
        cnt = jnp.sum((a_ref[...] >= midf).astype(jnp.int32), axis=1,
                      keepdims=True)
        nd = jnp.logical_not(done)
        hit = jnp.logical_and(cnt == K, nd)
        ge = cnt >= K
        nlo = jnp.where(jnp.logical_and(nd, ge), mid, lo)
        nhi = jnp.where(jnp.logical_and(nd, ge), hi, mid - 1)
        conv = nhi <= nlo
        t = tb_ref[...]
        t = jnp.where(hit, mid, t)
        t = jnp.where(jnp.logical_and(conv, jnp.logical_and(
            nd, jnp.logical_not(hit))), nlo, t)
        ndone = jnp.logical_or(done, jnp.logical_or(hit, conv))
        lo_ref[...] = nlo
        hi_ref[...] = nhi
        tb_ref[...] = t
        done_ref[...] = ndone.astype(jnp.int32)
        return jnp.logical_not(jnp.all(ndone))

    jax.lax.while_loop(cond, body, jnp.logical_not(jnp.all(done_ref[...] != 0)))
    t_ref[...] = jax.lax.bitcast_convert_type(tb_ref[...], jnp.float32)


def _thresholds(acts, m1):
    M, F = acts.shape
    bmt = min(BMT, M)
    return pl.pallas_call(
        _thresh_kernel,
        grid=(M // bmt,),
        in_specs=[
            pl.BlockSpec((bmt, F), lambda i: (i, 0)),
            pl.BlockSpec((bmt, F // G), lambda i: (i, 0)),
        ],
        out_specs=pl.BlockSpec((bmt, 1), lambda i: (i, 0)),
        out_shape=jax.ShapeDtypeStruct((M, 1), jnp.float32),
        scratch_shapes=[pltpu.VMEM((bmt, F // G // 8), jnp.float32)]
        + [pltpu.VMEM((bmt, 1), jnp.int32) for _ in range(4)],
    )(acts, m1)


def _dec_kernel(a_ref, w_ref, t_ref, bdec_ref, o_ref):
    j = pl.program_id(1)
    f = jnp.where(a_ref[...] >= t_ref[...], a_ref[...], 0.0)
    part = jax.lax.dot_general(f, w_ref[...], (((1,), (1,)), ((), ())),
                               preferred_element_type=jnp.float32)

    @pl.when(j == 0)
    def _():
        o_ref[...] = part + bdec_ref[...]

    @pl.when(j > 0)
    def _():
        o_ref[...] += part


def _decode(acts, thresh, W_dec, b_dec):
    M, F = acts.shape
    D = W_dec.shape[0]
    bmd = min(BMD, M)
    nj, ni = F // BND, M // bmd
    return pl.pallas_call(
        _dec_kernel,
        grid=(ni, nj),
        in_specs=[
            pl.BlockSpec((bmd, BND), lambda i, j: (i, j)),
            pl.BlockSpec((D, BND), lambda i, j: (0, j)),
            pl.BlockSpec((bmd, 1), lambda i, j: (i, 0)),
            pl.BlockSpec((1, D), lambda i, j: (0, 0)),
        ],
        out_specs=pl.BlockSpec((bmd, D), lambda i, j: (i, 0)),
        out_shape=jax.ShapeDtypeStruct((M, D), jnp.float32),
    )(acts, W_dec, thresh, b_dec.reshape(1, D))


def _ae(x, W_enc, b_enc, W_dec, b_dec):
    orig_shape = x.shape
    xf = x.reshape(-1, orig_shape[-1])
    acts, m1 = _encode_acts(xf, W_enc, b_enc, b_dec)
    thresh = _thresholds(acts, m1)
    recon = _decode(acts, thresh, W_dec, b_dec)
    return recon.reshape(orig_shape)


def kernel(x_attn_0, x_mlp_0,
           W_enc_attn_0, b_enc_attn_0, W_dec_attn_0, b_dec_attn_0,
           W_enc_mlp_0, b_enc_mlp_0, W_dec_mlp_0, b_dec_mlp_0):
    recon_attn = _ae(x_attn_0, W_enc_attn_0, b_enc_attn_0, W_dec_attn_0, b_dec_attn_0)
    recon_mlp = _ae(x_mlp_0, W_enc_mlp_0, b_enc_mlp_0, W_dec_mlp_0, b_dec_mlp_0)
    return (recon_attn, recon_mlp)
